# baseline (speedup 1.0000x reference)
import functools

import jax
import jax.numpy as jnp
from jax import lax
from jax.experimental import pallas as pl
from jax.experimental.pallas import tpu as pltpu
from jax.experimental.pallas import tpu_sc as plsc

NC = 2
NS = 16
NW = NC * NS

BATCH = 4096
SEQ = 200
EMB = 64
TOTAL = BATCH * SEQ
PER_W = TOTAL // NW
CHUNK = 256
CHUNKS = PER_W // CHUNK
PAIRS = CHUNK // 2


def _make_gather():
  mesh = plsc.VectorSubcoreMesh(
      core_axis_name="c", subcore_axis_name="s",
      num_cores=NC, num_subcores=NS)

  @functools.partial(
      pl.kernel,
      out_type=jax.ShapeDtypeStruct((TOTAL // 2, 2 * EMB), jnp.float32),
      mesh=mesh,
      scratch_types=[
          pltpu.VMEM((CHUNKS, CHUNK), jnp.int32),
          [pltpu.VMEM((CHUNK,), jnp.int32)] * 2,
          [pltpu.VMEM((CHUNK, 2 * EMB), jnp.float32)] * 2,
          [pltpu.VMEM((PAIRS, 2 * EMB), jnp.float32)] * 2,
          [pltpu.SemaphoreType.DMA] * 2,
          [pltpu.SemaphoreType.DMA] * 2,
      ],
      compiler_params=pltpu.CompilerParams(use_tc_tiling_on_sc=False),
  )
  def gather_kernel(word_hbm, table_hbm, out_hbm, idx_v, pidx, bufs, stages,
                    gsems, ssems):
    wid = lax.axis_index("s") * NC + lax.axis_index("c")
    pltpu.sync_copy(word_hbm.at[wid], idx_v)

    def fill(j, b):
      for k in range(CHUNK // 16):
        pidx[b][pl.ds(k * 16, 16)] = idx_v[j, pl.ds(k * 16, 16)] >> 1
      pltpu.async_copy(table_hbm.at[pidx[b]], bufs[b], gsems[b])

    def drain_fill(b):
      pltpu.make_async_copy(table_hbm.at[pidx[b]], bufs[b], gsems[b]).wait()

    def compact(j, b):
      def group(g, carry):
        k0 = g * 16
        srcs = (idx_v[j, pl.ds(k0, 16)] & 1) * EMB
        for l in range(16):
          k = k0 + l
          src = srcs[l]
          dst = (l & 1) * EMB
          kp = k >> 1
          for c in range(EMB // 16):
            stages[b][kp, pl.ds(dst + c * 16, 16)] = (
                bufs[b][k, pl.ds(src + c * 16, 16)])
        return carry
      lax.fori_loop(0, CHUNK // 16, group, 0)

    def store(j, b):
      base = (wid * CHUNKS + j) * PAIRS
      pltpu.async_copy(stages[b], out_hbm.at[pl.ds(base, PAIRS)], ssems[b])

    def wait_store(j, b):
      base = (wid * CHUNKS + j) * PAIRS
      pltpu.make_async_copy(stages[b], out_hbm.at[pl.ds(base, PAIRS)],
                            ssems[b]).wait()

    fill(0, 0)

    def outer(j0, carry):
      for b in range(2):
        j = j0 + b
        nb = 1 - b

        @pl.when(j + 1 < CHUNKS)
        def _():
          fill(j + 1, nb)

        drain_fill(b)

        @pl.when(j >= 2)
        def _():
          wait_store(j - 2, b)
        compact(j, b)
        store(j, b)
      return carry

    lax.fori_loop(0, CHUNKS // 2, lambda i, c: outer(i * 2, c), 0)

    wait_store(CHUNKS - 2, 0)
    wait_store(CHUNKS - 1, 1)

  return gather_kernel


_gather = _make_gather()


def kernel(WORD, word_table):
  idx = WORD.reshape(NW, CHUNKS, CHUNK)
  table2 = word_table.reshape(word_table.shape[0] // 2, 2 * EMB)
  out2 = _gather(idx, table2)
  return out2.reshape(TOTAL, EMB, 1)


# speedup vs baseline: 1.2634x; 1.2634x over previous
import functools

import jax
import jax.numpy as jnp
from jax import lax
from jax.experimental import pallas as pl
from jax.experimental.pallas import tpu as pltpu
from jax.experimental.pallas import tpu_sc as plsc

NC = 2
NS = 16
NW = NC * NS

BATCH = 4096
SEQ = 200
EMB = 64
TOTAL = BATCH * SEQ
PER_W = TOTAL // NW
CHUNK = 128
CHUNKS = PER_W // CHUNK
PAIRS = CHUNK // 2


def _make_gather():
  mesh = plsc.VectorSubcoreMesh(
      core_axis_name="c", subcore_axis_name="s",
      num_cores=NC, num_subcores=NS)

  @functools.partial(
      pl.kernel,
      out_type=jax.ShapeDtypeStruct((TOTAL, EMB), jnp.float32),
      mesh=mesh,
      scratch_types=[
          pltpu.VMEM((CHUNKS, CHUNK), jnp.int32),
          [pltpu.VMEM((CHUNK,), jnp.int32)] * 2,
          [pltpu.VMEM((CHUNK, 2 * EMB), jnp.float32)] * 2,
          [pltpu.VMEM((CHUNK, EMB), jnp.float32)] * 2,
          [pltpu.SemaphoreType.DMA] * 2,
          [pltpu.SemaphoreType.DMA] * 2,
      ],
      compiler_params=pltpu.CompilerParams(use_tc_tiling_on_sc=True),
  )
  def gather_kernel(word_hbm, table_hbm, out_hbm, idx_v, pidx, bufs, stages,
                    gsems, ssems):
    wid = lax.axis_index("s") * NC + lax.axis_index("c")
    pltpu.sync_copy(word_hbm.at[wid], idx_v)

    def fill(j, b):
      for k in range(CHUNK // 16):
        pidx[b][pl.ds(k * 16, 16)] = idx_v[j, pl.ds(k * 16, 16)] >> 1
      pltpu.async_copy(table_hbm.at[pidx[b]], bufs[b], gsems[b])

    def drain_fill(b):
      pltpu.make_async_copy(table_hbm.at[pidx[b]], bufs[b], gsems[b]).wait()

    def compact(j, b):
      def group(g, carry):
        k0 = g * 16
        srcs = (idx_v[j, pl.ds(k0, 16)] & 1) * EMB
        for l in range(16):
          k = k0 + l
          src = srcs[l]
          for c in range(EMB // 16):
            stages[b][k, pl.ds(c * 16, 16)] = (
                bufs[b][k, pl.ds(src + c * 16, 16)])
        return carry
      lax.fori_loop(0, CHUNK // 16, group, 0)

    def store(j, b):
      base = (wid * CHUNKS + j) * CHUNK
      pltpu.async_copy(stages[b], out_hbm.at[pl.ds(base, CHUNK)], ssems[b])

    def wait_store(j, b):
      base = (wid * CHUNKS + j) * CHUNK
      pltpu.make_async_copy(stages[b], out_hbm.at[pl.ds(base, CHUNK)],
                            ssems[b]).wait()

    fill(0, 0)

    def outer(j0, carry):
      for b in range(2):
        j = j0 + b
        nb = 1 - b

        @pl.when(j + 1 < CHUNKS)
        def _():
          fill(j + 1, nb)

        drain_fill(b)

        @pl.when(j >= 2)
        def _():
          wait_store(j - 2, b)
        compact(j, b)
        store(j, b)
      return carry

    lax.fori_loop(0, CHUNKS // 2, lambda i, c: outer(i * 2, c), 0)

    wait_store(CHUNKS - 2, 0)
    wait_store(CHUNKS - 1, 1)

  return gather_kernel


_gather = _make_gather()


def kernel(WORD, word_table):
  idx = WORD.reshape(NW, CHUNKS, CHUNK)
  table2 = word_table.reshape(word_table.shape[0] // 2, 2 * EMB)
  out = _gather(idx, table2)
  return out.reshape(TOTAL, EMB, 1)
